# Initial kernel scaffold; baseline (speedup 1.0000x reference)
import jax
import jax.numpy as jnp
from jax.experimental import pallas as pl


def kernel(features, ins_indices_batch, W1, b1, eca_w):
    raise NotImplementedError("write your pallas kernel here")


# speedup vs baseline: 12.2520x; 12.2520x over previous
import jax
import jax.numpy as jnp
from jax.experimental import pallas as pl

C = 128
I = 64
EPS = 1e-5
R = 2048


def _onehot(seg_ref):
    seg = seg_ref[0, 0, :]
    iota = jax.lax.broadcasted_iota(jnp.int32, (I, seg.shape[0]), 0)
    return (iota == seg[None, :]).astype(jnp.float32)


def _x_block(f_ref, W1_ref, b1_ref):
    return jnp.dot(f_ref[...], W1_ref[...],
                   preferred_element_type=jnp.float32) + b1_ref[...]


def _stats(ss_ref, cnt_ref):
    cnt = jnp.maximum(cnt_ref[...], 1.0)
    sums = ss_ref[:, :C]
    sq = ss_ref[:, C:]
    mean = sums / cnt
    var = sq / cnt - mean * mean
    rstd = jax.lax.rsqrt(var + EPS)
    return cnt, mean, rstd


def _phase1(f_ref, seg_ref, W1_ref, b1_ref, ss_ref, cnt_ref):
    b = pl.program_id(0)

    @pl.when(b == 0)
    def _():
        ss_ref[...] = jnp.zeros_like(ss_ref)
        cnt_ref[...] = jnp.zeros_like(cnt_ref)

    x = _x_block(f_ref, W1_ref, b1_ref)
    oh = _onehot(seg_ref)
    t = jnp.concatenate([x, x * x], axis=1)
    ss_ref[...] = ss_ref[...] + jax.lax.dot_general(
        oh, t, (((1,), (0,)), ((), ())), preferred_element_type=jnp.float32)
    cnt_ref[...] = cnt_ref[...] + jnp.sum(oh, axis=1, keepdims=True)


def _phase2(f_ref, seg_ref, W1_ref, b1_ref, ss_ref, cnt_ref, s2_ref):
    b = pl.program_id(0)

    @pl.when(b == 0)
    def _():
        s2_ref[...] = jnp.zeros_like(s2_ref)

    _, mean, rstd = _stats(ss_ref, cnt_ref)
    AB = jnp.concatenate([rstd, -mean * rstd], axis=1)
    x = _x_block(f_ref, W1_ref, b1_ref)
    oh = _onehot(seg_ref)
    rows = jax.lax.dot_general(
        oh, AB, (((0,), (0,)), ((), ())),
        preferred_element_type=jnp.float32)
    xr = jnp.maximum(x * rows[:, :C] + rows[:, C:], 0.0)
    s2_ref[...] = s2_ref[...] + jax.lax.dot_general(
        oh, xr, (((1,), (0,)), ((), ())), preferred_element_type=jnp.float32)


def _phase3(f_ref, seg_ref, W1_ref, b1_ref, ss_ref, cnt_ref, s2_ref, T_ref,
            out_ref):
    cnt, mean, rstd = _stats(ss_ref, cnt_ref)
    inst_mean = s2_ref[...] / cnt
    conv = jnp.dot(inst_mean, T_ref[...], preferred_element_type=jnp.float32)
    gate = jax.nn.sigmoid(conv)
    a = rstd * gate
    AB = jnp.concatenate([a, -mean * a], axis=1)
    x = _x_block(f_ref, W1_ref, b1_ref)
    oh = _onehot(seg_ref)
    rows = jax.lax.dot_general(
        oh, AB, (((0,), (0,)), ((), ())),
        preferred_element_type=jnp.float32)
    out_ref[...] = jnp.maximum(x * rows[:, :C] + rows[:, C:], 0.0)


def kernel(features, ins_indices_batch, W1, b1, eca_w):
    N = features.shape[0]
    NB = N // R
    seg3 = ins_indices_batch.reshape(NB, 1, R)
    b1r = b1.reshape(1, C)
    T = (eca_w[0] * jnp.eye(C, k=1) + eca_w[1] * jnp.eye(C)
         + eca_w[2] * jnp.eye(C, k=-1)).astype(jnp.float32)

    f_spec = pl.BlockSpec((R, C), lambda b: (b, 0))
    seg_spec = pl.BlockSpec((1, 1, R), lambda b: (b, 0, 0))
    w_spec = pl.BlockSpec((C, C), lambda b: (0, 0))
    b_spec = pl.BlockSpec((1, C), lambda b: (0, 0))
    acc2_spec = pl.BlockSpec((I, 2 * C), lambda b: (0, 0))
    acc1_spec = pl.BlockSpec((I, C), lambda b: (0, 0))

    ss, cnt = pl.pallas_call(
        _phase1,
        grid=(NB,),
        in_specs=[f_spec, seg_spec, w_spec, b_spec],
        out_specs=[acc2_spec, acc1_spec],
        out_shape=[jax.ShapeDtypeStruct((I, 2 * C), jnp.float32),
                   jax.ShapeDtypeStruct((I, C), jnp.float32)],
    )(features, seg3, W1, b1r)

    s2 = pl.pallas_call(
        _phase2,
        grid=(NB,),
        in_specs=[f_spec, seg_spec, w_spec, b_spec, acc2_spec, acc1_spec],
        out_specs=acc1_spec,
        out_shape=jax.ShapeDtypeStruct((I, C), jnp.float32),
    )(features, seg3, W1, b1r, ss, cnt)

    out = pl.pallas_call(
        _phase3,
        grid=(NB,),
        in_specs=[f_spec, seg_spec, w_spec, b_spec, acc2_spec, acc1_spec,
                  acc1_spec, w_spec],
        out_specs=f_spec,
        out_shape=jax.ShapeDtypeStruct((N, C), jnp.float32),
    )(features, seg3, W1, b1r, ss, cnt, s2, T)
    return out


# speedup vs baseline: 15.1063x; 1.2330x over previous
import jax
import jax.numpy as jnp
from jax.experimental import pallas as pl
from jax.experimental.pallas import tpu as pltpu

C = 128
I = 64
EPS = 1e-5
R = 2048
BF = jnp.bfloat16


def _onehot(seg_ref):
    seg = seg_ref[0, 0, :]
    iota = jax.lax.broadcasted_iota(jnp.int32, (I, seg.shape[0]), 0)
    return (iota == seg[None, :]).astype(BF)


def _stats(ss_ref, cnt_ref):
    cnt = jnp.maximum(cnt_ref[...], 1.0)
    mean = ss_ref[:, :C] / cnt
    var = ss_ref[:, C:] / cnt - mean * mean
    rstd = jax.lax.rsqrt(var + EPS)
    return cnt, mean, rstd


def _gather_rows(oh, ab):
    return jax.lax.dot_general(
        oh, ab.astype(BF), (((0,), (0,)), ((), ())),
        preferred_element_type=jnp.float32)


def _seg_sum(oh, vals):
    return jax.lax.dot_general(
        oh, vals.astype(BF), (((1,), (0,)), ((), ())),
        preferred_element_type=jnp.float32)


def _body(f_ref, seg_ref, W1_ref, b1_ref, T_ref, out_ref,
          xs_ref, ss_ref, cnt_ref, s2_ref):
    p = pl.program_id(0)
    b = pl.program_id(1)
    oh = _onehot(seg_ref)
    rows = pl.ds(b * R, R)

    @pl.when(jnp.logical_and(p == 0, b == 0))
    def _():
        ss_ref[...] = jnp.zeros_like(ss_ref)
        cnt_ref[...] = jnp.zeros_like(cnt_ref)
        s2_ref[...] = jnp.zeros_like(s2_ref)

    @pl.when(p == 0)
    def _():
        x = jnp.dot(f_ref[...].astype(BF), W1_ref[...].astype(BF),
                    preferred_element_type=jnp.float32) + b1_ref[...]
        xs_ref[rows, :] = x.astype(BF)
        ss_ref[...] = ss_ref[...] + _seg_sum(
            oh, jnp.concatenate([x, x * x], axis=1))
        cnt_ref[...] = cnt_ref[...] + jnp.sum(
            oh.astype(jnp.float32), axis=1, keepdims=True)

    @pl.when(p == 1)
    def _():
        _, mean, rstd = _stats(ss_ref, cnt_ref)
        ab = jnp.concatenate([rstd, -mean * rstd], axis=1)
        r = _gather_rows(oh, ab)
        x = xs_ref[rows, :].astype(jnp.float32)
        xr = jnp.maximum(x * r[:, :C] + r[:, C:], 0.0)
        xs_ref[rows, :] = xr.astype(BF)
        s2_ref[...] = s2_ref[...] + _seg_sum(oh, xr)

    @pl.when(p == 2)
    def _():
        cnt, _, _ = _stats(ss_ref, cnt_ref)
        inst_mean = s2_ref[...] / cnt
        conv = jnp.dot(inst_mean, T_ref[...],
                       preferred_element_type=jnp.float32)
        gate = jax.nn.sigmoid(conv)
        g = _gather_rows(oh, gate)
        xr = xs_ref[rows, :].astype(jnp.float32)
        out_ref[...] = xr * g


def kernel(features, ins_indices_batch, W1, b1, eca_w):
    N = features.shape[0]
    NB = N // R
    seg3 = ins_indices_batch.reshape(NB, 1, R)
    b1r = b1.reshape(1, C)
    T = (eca_w[0] * jnp.eye(C, k=1) + eca_w[1] * jnp.eye(C)
         + eca_w[2] * jnp.eye(C, k=-1)).astype(jnp.float32)

    return pl.pallas_call(
        _body,
        grid=(3, NB),
        in_specs=[
            pl.BlockSpec((R, C), lambda p, b: (jnp.where(p == 0, b, 0), 0)),
            pl.BlockSpec((1, 1, R), lambda p, b: (b, 0, 0)),
            pl.BlockSpec((C, C), lambda p, b: (0, 0)),
            pl.BlockSpec((1, C), lambda p, b: (0, 0)),
            pl.BlockSpec((C, C), lambda p, b: (0, 0)),
        ],
        out_specs=pl.BlockSpec((R, C), lambda p, b: (jnp.where(p == 2, b, 0), 0)),
        out_shape=jax.ShapeDtypeStruct((N, C), jnp.float32),
        scratch_shapes=[
            pltpu.VMEM((N, C), BF),
            pltpu.VMEM((I, 2 * C), jnp.float32),
            pltpu.VMEM((I, C), jnp.float32),
            pltpu.VMEM((I, C), jnp.float32),
        ],
    )(features, seg3, W1, b1r, T)


# speedup vs baseline: 17.1073x; 1.1325x over previous
import jax
import jax.numpy as jnp
from jax.experimental import pallas as pl
from jax.experimental.pallas import tpu as pltpu

C = 128
I = 64
EPS = 1e-5
R = 2048
BF = jnp.bfloat16
F32 = jnp.float32


def _gather_rows(oh, ab):
    return jax.lax.dot_general(
        oh, ab.astype(BF), (((0,), (0,)), ((), ())),
        preferred_element_type=F32)


def _seg_sum(oh, vals):
    return jax.lax.dot_general(
        oh, vals.astype(BF), (((1,), (0,)), ((), ())),
        preferred_element_type=F32)


def _outer(f_hbm, seg_ref, W1_ref, b1_ref, T_ref, out_hbm,
           xs_ref, ss_ref, cnt_ref, s2_ref):
    nb = seg_ref.shape[0]

    def onehot(b):
        seg = seg_ref[b][0, :]
        iota = jax.lax.broadcasted_iota(jnp.int32, (I, R), 0)
        return (iota == seg[None, :]).astype(BF)

    ss_ref[...] = jnp.zeros_like(ss_ref)
    cnt_ref[...] = jnp.zeros_like(cnt_ref)
    s2_ref[...] = jnp.zeros_like(s2_ref)

    W1b = W1_ref[...].astype(BF)
    b1v = b1_ref[...]

    def l0(idx, f_blk):
        b = idx[0]
        x = jnp.dot(f_blk[...].astype(BF), W1b,
                    preferred_element_type=F32) + b1v
        xs_ref[pl.ds(b * R, R), :] = x.astype(BF)
        oh = onehot(b)
        ss_ref[...] = ss_ref[...] + _seg_sum(
            oh, jnp.concatenate([x, x * x], axis=1))
        cnt_ref[...] = cnt_ref[...] + jnp.sum(
            oh.astype(F32), axis=1, keepdims=True)

    pltpu.emit_pipeline(
        l0, grid=(nb,),
        in_specs=[pl.BlockSpec((R, C), lambda b: (b, 0))],
        _explicit_indices=True,
    )(f_hbm)

    cnt = jnp.maximum(cnt_ref[...], 1.0)
    mean = ss_ref[:, :C] / cnt
    var = ss_ref[:, C:] / cnt - mean * mean
    rstd = jax.lax.rsqrt(var + EPS)
    ab = jnp.concatenate([rstd, -mean * rstd], axis=1).astype(BF)

    def l1(b, _):
        oh = onehot(b)
        r = _gather_rows(oh, ab)
        x = xs_ref[pl.ds(b * R, R), :].astype(F32)
        xr = jnp.maximum(x * r[:, :C] + r[:, C:], 0.0)
        xs_ref[pl.ds(b * R, R), :] = xr.astype(BF)
        s2_ref[...] = s2_ref[...] + _seg_sum(oh, xr)
        return 0

    jax.lax.fori_loop(0, nb, l1, 0)

    inst_mean = s2_ref[...] / cnt
    conv = jnp.dot(inst_mean, T_ref[...], preferred_element_type=F32)
    gate = jax.nn.sigmoid(conv).astype(BF)

    def l2(idx, out_blk):
        b = idx[0]
        g = _gather_rows(onehot(b), gate)
        out_blk[...] = xs_ref[pl.ds(b * R, R), :].astype(F32) * g

    pltpu.emit_pipeline(
        l2, grid=(nb,),
        out_specs=[pl.BlockSpec((R, C), lambda b: (b, 0))],
        _explicit_indices=True,
    )(out_hbm)


def kernel(features, ins_indices_batch, W1, b1, eca_w):
    N = features.shape[0]
    NB = N // R
    seg3 = ins_indices_batch.reshape(NB, 1, R)
    b1r = b1.reshape(1, C)
    T = (eca_w[0] * jnp.eye(C, k=1) + eca_w[1] * jnp.eye(C)
         + eca_w[2] * jnp.eye(C, k=-1)).astype(F32)

    return pl.pallas_call(
        _outer,
        in_specs=[
            pl.BlockSpec(memory_space=pl.MemorySpace.ANY),
            pl.BlockSpec(memory_space=pltpu.VMEM),
            pl.BlockSpec(memory_space=pltpu.VMEM),
            pl.BlockSpec(memory_space=pltpu.VMEM),
            pl.BlockSpec(memory_space=pltpu.VMEM),
        ],
        out_specs=pl.BlockSpec(memory_space=pl.MemorySpace.ANY),
        out_shape=jax.ShapeDtypeStruct((N, C), F32),
        scratch_shapes=[
            pltpu.VMEM((N, C), BF),
            pltpu.VMEM((I, 2 * C), F32),
            pltpu.VMEM((I, C), F32),
            pltpu.VMEM((I, C), F32),
        ],
    )(features, seg3, W1, b1r, T)


# speedup vs baseline: 23.6241x; 1.3809x over previous
import jax
import jax.numpy as jnp
from jax.experimental import pallas as pl
from jax.experimental.pallas import tpu as pltpu

C = 128
I = 64
EPS = 1e-5
R = 4096
BF = jnp.bfloat16
F32 = jnp.float32


def _gather_rows(oh, ab):
    return jax.lax.dot_general(
        oh, ab.astype(BF), (((0,), (0,)), ((), ())),
        preferred_element_type=F32)


def _seg_sum(oh, vals):
    return jax.lax.dot_general(
        oh, vals.astype(BF), (((1,), (0,)), ((), ())),
        preferred_element_type=F32)


def _outer(f_hbm, seg_ref, W1_ref, b1_ref, T_ref, out_hbm,
           xs_ref, ss_ref, cnt_ref, s2_ref):
    nb = seg_ref.shape[0]

    def onehot(b):
        seg = seg_ref[b][0, :]
        iota = jax.lax.broadcasted_iota(jnp.int32, (I, R), 0)
        return (iota == seg[None, :]).astype(BF)

    ss_ref[...] = jnp.zeros_like(ss_ref)
    cnt_ref[...] = jnp.zeros_like(cnt_ref)
    s2_ref[...] = jnp.zeros_like(s2_ref)

    W1b = W1_ref[...].astype(BF)
    b1v = b1_ref[...]

    def l0(idx, f_blk):
        b = idx[0]
        x = jnp.dot(f_blk[...].astype(BF), W1b,
                    preferred_element_type=F32) + b1v
        xs_ref[pl.ds(b * R, R), :] = x.astype(BF)
        oh = onehot(b)
        ss_ref[...] = ss_ref[...] + _seg_sum(
            oh, jnp.concatenate([x, x * x], axis=1))
        cnt_ref[...] = cnt_ref[...] + jnp.sum(
            oh.astype(F32), axis=1, keepdims=True)

    pltpu.emit_pipeline(
        l0, grid=(nb,),
        in_specs=[pl.BlockSpec((R, C), lambda b: (b, 0))],
        _explicit_indices=True,
    )(f_hbm)

    cnt = jnp.maximum(cnt_ref[...], 1.0)
    mean = ss_ref[:, :C] / cnt
    var = ss_ref[:, C:] / cnt - mean * mean
    rstd = jax.lax.rsqrt(var + EPS)
    ab = jnp.concatenate([rstd, -mean * rstd], axis=1).astype(BF)

    def l1(b, _):
        oh = onehot(b)
        r = _gather_rows(oh, ab)
        x = xs_ref[pl.ds(b * R, R), :].astype(F32)
        xr = jnp.maximum(x * r[:, :C] + r[:, C:], 0.0)
        xs_ref[pl.ds(b * R, R), :] = xr.astype(BF)
        s2_ref[...] = s2_ref[...] + _seg_sum(oh, xr)
        return 0

    jax.lax.fori_loop(0, nb, l1, 0)

    inst_mean = s2_ref[...] / cnt
    conv = jnp.dot(inst_mean, T_ref[...], preferred_element_type=F32)
    gate = jax.nn.sigmoid(conv).astype(BF)

    def l2(idx, out_blk):
        b = idx[0]
        g = _gather_rows(onehot(b), gate)
        out_blk[...] = xs_ref[pl.ds(b * R, R), :].astype(F32) * g

    pltpu.emit_pipeline(
        l2, grid=(nb,),
        out_specs=[pl.BlockSpec((R, C), lambda b: (b, 0))],
        _explicit_indices=True,
    )(out_hbm)


def kernel(features, ins_indices_batch, W1, b1, eca_w):
    N = features.shape[0]
    NB = N // R
    seg3 = ins_indices_batch.reshape(NB, 1, R)
    b1r = b1.reshape(1, C)
    T = (eca_w[0] * jnp.eye(C, k=1) + eca_w[1] * jnp.eye(C)
         + eca_w[2] * jnp.eye(C, k=-1)).astype(F32)

    return pl.pallas_call(
        _outer,
        in_specs=[
            pl.BlockSpec(memory_space=pl.MemorySpace.ANY),
            pl.BlockSpec(memory_space=pltpu.VMEM),
            pl.BlockSpec(memory_space=pltpu.VMEM),
            pl.BlockSpec(memory_space=pltpu.VMEM),
            pl.BlockSpec(memory_space=pltpu.VMEM),
        ],
        out_specs=pl.BlockSpec(memory_space=pl.MemorySpace.ANY),
        out_shape=jax.ShapeDtypeStruct((N, C), F32),
        scratch_shapes=[
            pltpu.VMEM((N, C), BF),
            pltpu.VMEM((I, 2 * C), F32),
            pltpu.VMEM((I, C), F32),
            pltpu.VMEM((I, C), F32),
        ],
    )(features, seg3, W1, b1r, T)


# speedup vs baseline: 29.0772x; 1.2308x over previous
import jax
import jax.numpy as jnp
from jax.experimental import pallas as pl
from jax.experimental.pallas import tpu as pltpu

C = 128
I = 64
EPS = 1e-5
R = 8192
BF = jnp.bfloat16
F32 = jnp.float32


def _gather_rows(oh, ab):
    return jax.lax.dot_general(
        oh, ab.astype(BF), (((0,), (0,)), ((), ())),
        preferred_element_type=F32)


def _seg_sum(oh, vals):
    return jax.lax.dot_general(
        oh, vals.astype(BF), (((1,), (0,)), ((), ())),
        preferred_element_type=F32)


def _outer(f_hbm, seg_ref, W1_ref, b1_ref, T_ref, out_hbm,
           xs_ref, ss_ref, cnt_ref, s2_ref):
    nb = seg_ref.shape[0]

    def onehot(b):
        seg = seg_ref[b][0, :]
        iota = jax.lax.broadcasted_iota(jnp.int32, (I, R), 0)
        return (iota == seg[None, :]).astype(BF)

    ss_ref[...] = jnp.zeros_like(ss_ref)
    cnt_ref[...] = jnp.zeros_like(cnt_ref)
    s2_ref[...] = jnp.zeros_like(s2_ref)

    W1b = W1_ref[...].astype(BF)
    b1v = b1_ref[...]

    def l0(idx, f_blk):
        b = idx[0]
        x = jnp.dot(f_blk[...].astype(BF), W1b,
                    preferred_element_type=F32) + b1v
        xs_ref[pl.ds(b * R, R), :] = x.astype(BF)
        oh = onehot(b)
        ss_ref[...] = ss_ref[...] + _seg_sum(
            oh, jnp.concatenate([x, x * x], axis=1))
        cnt_ref[...] = cnt_ref[...] + jnp.sum(
            oh.astype(F32), axis=1, keepdims=True)

    pltpu.emit_pipeline(
        l0, grid=(nb,),
        in_specs=[pl.BlockSpec((R, C), lambda b: (b, 0))],
        _explicit_indices=True,
    )(f_hbm)

    cnt = jnp.maximum(cnt_ref[...], 1.0)
    mean = ss_ref[:, :C] / cnt
    var = ss_ref[:, C:] / cnt - mean * mean
    rstd = jax.lax.rsqrt(var + EPS)
    ab = jnp.concatenate([rstd, -mean * rstd], axis=1).astype(BF)

    def l1(b, _):
        oh = onehot(b)
        r = _gather_rows(oh, ab)
        x = xs_ref[pl.ds(b * R, R), :].astype(F32)
        xr = jnp.maximum(x * r[:, :C] + r[:, C:], 0.0)
        xs_ref[pl.ds(b * R, R), :] = xr.astype(BF)
        s2_ref[...] = s2_ref[...] + _seg_sum(oh, xr)
        return 0

    jax.lax.fori_loop(0, nb, l1, 0)

    inst_mean = s2_ref[...] / cnt
    conv = jnp.dot(inst_mean, T_ref[...], preferred_element_type=F32)
    gate = jax.nn.sigmoid(conv).astype(BF)

    def l2(idx, out_blk):
        b = idx[0]
        g = _gather_rows(onehot(b), gate)
        out_blk[...] = xs_ref[pl.ds(b * R, R), :].astype(F32) * g

    pltpu.emit_pipeline(
        l2, grid=(nb,),
        out_specs=[pl.BlockSpec((R, C), lambda b: (b, 0))],
        _explicit_indices=True,
    )(out_hbm)


def kernel(features, ins_indices_batch, W1, b1, eca_w):
    N = features.shape[0]
    NB = N // R
    seg3 = ins_indices_batch.reshape(NB, 1, R)
    b1r = b1.reshape(1, C)
    T = (eca_w[0] * jnp.eye(C, k=1) + eca_w[1] * jnp.eye(C)
         + eca_w[2] * jnp.eye(C, k=-1)).astype(F32)

    return pl.pallas_call(
        _outer,
        in_specs=[
            pl.BlockSpec(memory_space=pl.MemorySpace.ANY),
            pl.BlockSpec(memory_space=pltpu.VMEM),
            pl.BlockSpec(memory_space=pltpu.VMEM),
            pl.BlockSpec(memory_space=pltpu.VMEM),
            pl.BlockSpec(memory_space=pltpu.VMEM),
        ],
        out_specs=pl.BlockSpec(memory_space=pl.MemorySpace.ANY),
        out_shape=jax.ShapeDtypeStruct((N, C), F32),
        scratch_shapes=[
            pltpu.VMEM((N, C), BF),
            pltpu.VMEM((I, 2 * C), F32),
            pltpu.VMEM((I, C), F32),
            pltpu.VMEM((I, C), F32),
        ],
    )(features, seg3, W1, b1r, T)


# speedup vs baseline: 31.6007x; 1.0868x over previous
import jax
import jax.numpy as jnp
from jax.experimental import pallas as pl
from jax.experimental.pallas import tpu as pltpu

C = 128
I = 64
EPS = 1e-5
R = 16384
BF = jnp.bfloat16
F32 = jnp.float32


def _gather_rows(oh, ab):
    return jax.lax.dot_general(
        oh, ab.astype(BF), (((0,), (0,)), ((), ())),
        preferred_element_type=F32)


def _seg_sum(oh, vals):
    return jax.lax.dot_general(
        oh, vals.astype(BF), (((1,), (0,)), ((), ())),
        preferred_element_type=F32)


def _outer(f_hbm, seg_ref, W1_ref, b1_ref, T_ref, out_hbm,
           xs_ref, ss_ref, cnt_ref, s2_ref):
    nb = seg_ref.shape[0]

    def onehot(b):
        seg = seg_ref[b][0, :]
        iota = jax.lax.broadcasted_iota(jnp.int32, (I, R), 0)
        return (iota == seg[None, :]).astype(BF)

    ss_ref[...] = jnp.zeros_like(ss_ref)
    cnt_ref[...] = jnp.zeros_like(cnt_ref)
    s2_ref[...] = jnp.zeros_like(s2_ref)

    W1b = W1_ref[...].astype(BF)
    b1v = b1_ref[...]

    def l0(idx, f_blk):
        b = idx[0]
        x = jnp.dot(f_blk[...].astype(BF), W1b,
                    preferred_element_type=F32) + b1v
        xs_ref[pl.ds(b * R, R), :] = x.astype(BF)
        oh = onehot(b)
        ss_ref[...] = ss_ref[...] + _seg_sum(
            oh, jnp.concatenate([x, x * x], axis=1))
        cnt_ref[...] = cnt_ref[...] + jnp.sum(
            oh.astype(F32), axis=1, keepdims=True)

    pltpu.emit_pipeline(
        l0, grid=(nb,),
        in_specs=[pl.BlockSpec((R, C), lambda b: (b, 0))],
        _explicit_indices=True,
    )(f_hbm)

    cnt = jnp.maximum(cnt_ref[...], 1.0)
    mean = ss_ref[:, :C] / cnt
    var = ss_ref[:, C:] / cnt - mean * mean
    rstd = jax.lax.rsqrt(var + EPS)
    ab = jnp.concatenate([rstd, -mean * rstd], axis=1).astype(BF)

    def l1(b, _):
        oh = onehot(b)
        r = _gather_rows(oh, ab)
        x = xs_ref[pl.ds(b * R, R), :].astype(F32)
        xr = jnp.maximum(x * r[:, :C] + r[:, C:], 0.0)
        xs_ref[pl.ds(b * R, R), :] = xr.astype(BF)
        s2_ref[...] = s2_ref[...] + _seg_sum(oh, xr)
        return 0

    jax.lax.fori_loop(0, nb, l1, 0)

    inst_mean = s2_ref[...] / cnt
    conv = jnp.dot(inst_mean, T_ref[...], preferred_element_type=F32)
    gate = jax.nn.sigmoid(conv).astype(BF)

    def l2(idx, out_blk):
        b = idx[0]
        g = _gather_rows(onehot(b), gate)
        out_blk[...] = xs_ref[pl.ds(b * R, R), :].astype(F32) * g

    pltpu.emit_pipeline(
        l2, grid=(nb,),
        out_specs=[pl.BlockSpec((R, C), lambda b: (b, 0))],
        _explicit_indices=True,
    )(out_hbm)


def kernel(features, ins_indices_batch, W1, b1, eca_w):
    N = features.shape[0]
    NB = N // R
    seg3 = ins_indices_batch.reshape(NB, 1, R)
    b1r = b1.reshape(1, C)
    T = (eca_w[0] * jnp.eye(C, k=1) + eca_w[1] * jnp.eye(C)
         + eca_w[2] * jnp.eye(C, k=-1)).astype(F32)

    return pl.pallas_call(
        _outer,
        in_specs=[
            pl.BlockSpec(memory_space=pl.MemorySpace.ANY),
            pl.BlockSpec(memory_space=pltpu.VMEM),
            pl.BlockSpec(memory_space=pltpu.VMEM),
            pl.BlockSpec(memory_space=pltpu.VMEM),
            pl.BlockSpec(memory_space=pltpu.VMEM),
        ],
        out_specs=pl.BlockSpec(memory_space=pl.MemorySpace.ANY),
        out_shape=jax.ShapeDtypeStruct((N, C), F32),
        scratch_shapes=[
            pltpu.VMEM((N, C), BF),
            pltpu.VMEM((I, 2 * C), F32),
            pltpu.VMEM((I, C), F32),
            pltpu.VMEM((I, C), F32),
        ],
    )(features, seg3, W1, b1r, T)


# speedup vs baseline: 35.0085x; 1.1078x over previous
import jax
import jax.numpy as jnp
from jax.experimental import pallas as pl
from jax.experimental.pallas import tpu as pltpu

C = 128
I = 64
EPS = 1e-5
R = 16384
BF = jnp.bfloat16
F32 = jnp.float32


def _gather_rows(oh, ab):
    return jax.lax.dot_general(
        oh, ab.astype(BF), (((0,), (0,)), ((), ())),
        preferred_element_type=F32)


def _seg_sum(oh, vals):
    return jax.lax.dot_general(
        oh, vals.astype(BF), (((1,), (0,)), ((), ())),
        preferred_element_type=F32)


def _outer(f_hbm, seg_ref, W1_ref, b1_ref, T_ref, out_hbm,
           xs_ref, ss_ref, cnt_ref, s2_ref):
    nb = seg_ref.shape[0]

    def onehot(b):
        seg = seg_ref[b][0, :]
        iota = jax.lax.broadcasted_iota(jnp.int32, (I, R), 0)
        return (iota == seg[None, :]).astype(BF)

    ss_ref[...] = jnp.zeros_like(ss_ref)
    cnt_ref[...] = jnp.zeros_like(cnt_ref)
    s2_ref[...] = jnp.zeros_like(s2_ref)

    W1b = W1_ref[...].astype(BF)
    b1v = b1_ref[...]

    def l0(idx, f_blk):
        b = idx[0]
        x = jnp.dot(f_blk[...].astype(BF), W1b,
                    preferred_element_type=F32) + b1v
        xb = x.astype(BF)
        xs_ref[pl.ds(b * R, R), :] = xb
        oh = onehot(b)
        ss_ref[...] = ss_ref[...] + _seg_sum(
            oh, jnp.concatenate([xb, xb * xb], axis=1))
        cnt_ref[...] = cnt_ref[...] + jnp.sum(
            oh.astype(F32), axis=1, keepdims=True)

    pltpu.emit_pipeline(
        l0, grid=(nb,),
        in_specs=[pl.BlockSpec((R, C), lambda b: (b, 0))],
        _explicit_indices=True,
    )(f_hbm)

    cnt = jnp.maximum(cnt_ref[...], 1.0)
    mean = ss_ref[:, :C] / cnt
    var = ss_ref[:, C:] / cnt - mean * mean
    rstd = jax.lax.rsqrt(var + EPS)
    meanb = mean.astype(BF)

    def l1(b, _):
        oh = onehot(b)
        m = _gather_rows(oh, meanb)
        x = xs_ref[pl.ds(b * R, R), :].astype(F32)
        yb = jnp.maximum(x - m, 0.0).astype(BF)
        xs_ref[pl.ds(b * R, R), :] = yb
        s2_ref[...] = s2_ref[...] + _seg_sum(oh, yb)
        return 0

    jax.lax.fori_loop(0, nb, l1, 0)

    inst_mean = rstd * s2_ref[...] / cnt
    conv = jnp.dot(inst_mean, T_ref[...], preferred_element_type=F32)
    gate = jax.nn.sigmoid(conv)
    rg = (rstd * gate).astype(BF)

    def l2(idx, out_blk):
        b = idx[0]
        g = _gather_rows(onehot(b), rg)
        out_blk[...] = xs_ref[pl.ds(b * R, R), :].astype(F32) * g

    pltpu.emit_pipeline(
        l2, grid=(nb,),
        out_specs=[pl.BlockSpec((R, C), lambda b: (b, 0))],
        _explicit_indices=True,
    )(out_hbm)


def kernel(features, ins_indices_batch, W1, b1, eca_w):
    N = features.shape[0]
    NB = N // R
    seg3 = ins_indices_batch.reshape(NB, 1, R)
    b1r = b1.reshape(1, C)
    T = (eca_w[0] * jnp.eye(C, k=1) + eca_w[1] * jnp.eye(C)
         + eca_w[2] * jnp.eye(C, k=-1)).astype(F32)

    return pl.pallas_call(
        _outer,
        in_specs=[
            pl.BlockSpec(memory_space=pl.MemorySpace.ANY),
            pl.BlockSpec(memory_space=pltpu.VMEM),
            pl.BlockSpec(memory_space=pltpu.VMEM),
            pl.BlockSpec(memory_space=pltpu.VMEM),
            pl.BlockSpec(memory_space=pltpu.VMEM),
        ],
        out_specs=pl.BlockSpec(memory_space=pl.MemorySpace.ANY),
        out_shape=jax.ShapeDtypeStruct((N, C), F32),
        scratch_shapes=[
            pltpu.VMEM((N, C), BF),
            pltpu.VMEM((I, 2 * C), F32),
            pltpu.VMEM((I, C), F32),
            pltpu.VMEM((I, C), F32),
        ],
    )(features, seg3, W1, b1r, T)


# speedup vs baseline: 37.3159x; 1.0659x over previous
import jax
import jax.numpy as jnp
from jax.experimental import pallas as pl
from jax.experimental.pallas import tpu as pltpu

C = 128
I = 64
EPS = 1e-5
R = 16384
BF = jnp.bfloat16
F8 = jnp.float8_e4m3fn
F32 = jnp.float32


def _gather_rows(oh, ab):
    return jax.lax.dot_general(
        oh, ab.astype(BF), (((0,), (0,)), ((), ())),
        preferred_element_type=F32)


def _seg_sum(oh8, vals8):
    return jax.lax.dot_general(
        oh8, vals8, (((1,), (0,)), ((), ())),
        preferred_element_type=F32)


def _outer(f_hbm, seg_ref, W1_ref, b1_ref, T_ref, out_hbm,
           xs_ref, ss_ref, cnt_ref, s2_ref):
    nb = seg_ref.shape[0]

    def onehot_mask(b):
        seg = seg_ref[b][0, :]
        iota = jax.lax.broadcasted_iota(jnp.int32, (I, R), 0)
        return iota == seg[None, :]

    ss_ref[...] = jnp.zeros_like(ss_ref)
    cnt_ref[...] = jnp.zeros_like(cnt_ref)
    s2_ref[...] = jnp.zeros_like(s2_ref)

    W1b = W1_ref[...].astype(BF)
    b1v = b1_ref[...]

    def l0(idx, f_blk):
        b = idx[0]
        x = jnp.dot(f_blk[...].astype(BF), W1b,
                    preferred_element_type=F32) + b1v
        xb = x.astype(BF)
        xs_ref[pl.ds(b * R, R), :] = xb
        m = onehot_mask(b)
        oh8 = m.astype(F8)
        t8 = jnp.concatenate([xb, xb * xb], axis=1).astype(F8)
        ss_ref[...] = ss_ref[...] + _seg_sum(oh8, t8)
        cnt_ref[...] = cnt_ref[...] + jnp.sum(
            m.astype(F32), axis=1, keepdims=True)

    pltpu.emit_pipeline(
        l0, grid=(nb,),
        in_specs=[pl.BlockSpec((R, C), lambda b: (b, 0))],
        _explicit_indices=True,
    )(f_hbm)

    cnt = jnp.maximum(cnt_ref[...], 1.0)
    mean = ss_ref[:, :C] / cnt
    var = ss_ref[:, C:] / cnt - mean * mean
    rstd = jax.lax.rsqrt(var + EPS)
    meanb = mean.astype(BF)

    def l1(b, _):
        m = onehot_mask(b)
        mu = _gather_rows(m.astype(BF), meanb)
        x = xs_ref[pl.ds(b * R, R), :].astype(F32)
        yb = jnp.maximum(x - mu, 0.0).astype(BF)
        xs_ref[pl.ds(b * R, R), :] = yb
        s2_ref[...] = s2_ref[...] + _seg_sum(m.astype(F8), yb.astype(F8))
        return 0

    jax.lax.fori_loop(0, nb, l1, 0)

    inst_mean = rstd * s2_ref[...] / cnt
    conv = jnp.dot(inst_mean, T_ref[...], preferred_element_type=F32)
    gate = jax.nn.sigmoid(conv)
    rg = (rstd * gate).astype(BF)

    def l2(idx, out_blk):
        b = idx[0]
        g = _gather_rows(onehot_mask(b).astype(BF), rg)
        out_blk[...] = xs_ref[pl.ds(b * R, R), :].astype(F32) * g

    pltpu.emit_pipeline(
        l2, grid=(nb,),
        out_specs=[pl.BlockSpec((R, C), lambda b: (b, 0))],
        _explicit_indices=True,
    )(out_hbm)


def kernel(features, ins_indices_batch, W1, b1, eca_w):
    N = features.shape[0]
    NB = N // R
    seg3 = ins_indices_batch.reshape(NB, 1, R)
    b1r = b1.reshape(1, C)
    T = (eca_w[0] * jnp.eye(C, k=1) + eca_w[1] * jnp.eye(C)
         + eca_w[2] * jnp.eye(C, k=-1)).astype(F32)

    return pl.pallas_call(
        _outer,
        in_specs=[
            pl.BlockSpec(memory_space=pl.MemorySpace.ANY),
            pl.BlockSpec(memory_space=pltpu.VMEM),
            pl.BlockSpec(memory_space=pltpu.VMEM),
            pl.BlockSpec(memory_space=pltpu.VMEM),
            pl.BlockSpec(memory_space=pltpu.VMEM),
        ],
        out_specs=pl.BlockSpec(memory_space=pl.MemorySpace.ANY),
        out_shape=jax.ShapeDtypeStruct((N, C), F32),
        scratch_shapes=[
            pltpu.VMEM((N, C), BF),
            pltpu.VMEM((I, 2 * C), F32),
            pltpu.VMEM((I, C), F32),
            pltpu.VMEM((I, C), F32),
        ],
    )(features, seg3, W1, b1r, T)
